# baseline (speedup 1.0000x reference)
import functools

import jax
import jax.numpy as jnp
from jax import lax
from jax.experimental import pallas as pl
from jax.experimental.pallas import tpu as pltpu
from jax.experimental.pallas import tpu_sc as plsc

VOCAB = 30522
H = 768
L = 16
NV = H // L
NC, NS = 2, 16
NW = NC * NS
B, S = 4, 2048
T = B * S
PER_W = T // NW
CH = 64
NCHUNK = PER_W // CH
EPS = 1e-12


def _sc_body(ids_hbm, tt_hbm, word_hbm, pos_hbm, tok_hbm, gam_hbm, bet_hbm,
             out_hbm, idx_v, x_v, pos_v, tok_v, gam_v, bet_v, tt_v, sem):
    wid = lax.axis_index("s") * NC + lax.axis_index("c")
    base0 = wid * PER_W

    pltpu.sync_copy(tok_hbm, tok_v)
    pltpu.sync_copy(gam_hbm, gam_v)
    pltpu.sync_copy(bet_hbm, bet_v)
    for j in range(NV):
        sl = pl.ds(j * L, L)
        tok_v[1, sl] = tok_v[1, sl] - tok_v[0, sl]

    inv_h = jnp.float32(1.0 / H)

    def chunk_body(c, carry):
        base = base0 + c * CH
        pltpu.sync_copy(ids_hbm.at[pl.ds(base, CH)], idx_v)
        pltpu.sync_copy(tt_hbm.at[pl.ds(base, CH)], tt_v)
        pos_base = lax.rem(base, S)
        cp_pos = pltpu.async_copy(pos_hbm.at[pl.ds(pos_base, CH)], pos_v, sem)
        cp_word = pltpu.async_copy(word_hbm.at[idx_v], x_v, sem)
        cp_pos.wait()
        cp_word.wait()

        def tok_body(i, tcarry):
            ttg = tt_v[pl.ds((i // L) * L, L)].astype(jnp.float32)
            lane = i % L
            ttf = lax.gather(
                ttg, jnp.full((L, 1), 0, jnp.int32) + lane,
                lax.GatherDimensionNumbers(offset_dims=(),
                                           collapsed_slice_dims=(0,),
                                           start_index_map=(0,)),
                slice_sizes=(1,),
                mode=lax.GatherScatterMode.PROMISE_IN_BOUNDS)
            acc = jnp.zeros((L,), jnp.float32)
            acc2 = jnp.zeros((L,), jnp.float32)
            for j in range(NV):
                sl = pl.ds(j * L, L)
                x = (x_v[i, sl] + pos_v[i, sl]) + (tok_v[0, sl] + ttf * tok_v[1, sl])
                x_v[i, sl] = x
                acc = acc + x
                acc2 = acc2 + x * x
            s1 = jnp.sum(acc)
            s2 = jnp.sum(acc2)
            mean = s1 * inv_h
            var = s2 * inv_h - mean * mean
            v = jnp.full((L,), var + EPS, jnp.float32)
            bits = plsc.bitcast(v, jnp.int32)
            seed = jnp.full((L,), 0x5F3759DF, jnp.int32) - (bits >> 1)
            y = plsc.bitcast(seed, jnp.float32)
            for _ in range(3):
                y = y * (1.5 - 0.5 * v * y * y)
            meanv = jnp.full((L,), mean, jnp.float32)
            for j in range(NV):
                sl = pl.ds(j * L, L)
                xn = (x_v[i, sl] - meanv) * y
                x_v[i, sl] = xn * gam_v[sl] + bet_v[sl]
            return tcarry

        lax.fori_loop(0, CH, tok_body, 0, unroll=False)
        pltpu.sync_copy(x_v, out_hbm.at[pl.ds(base, CH)])
        return carry

    lax.fori_loop(0, NCHUNK, chunk_body, 0, unroll=False)


@functools.partial(jax.jit, static_argnames=())
def _run(ids, tts, word_emb, pos_emb, tok_type_emb, ln_gamma, ln_beta):
    mesh = plsc.VectorSubcoreMesh(core_axis_name="c", subcore_axis_name="s",
                                  num_cores=NC, num_subcores=NS)
    f = pl.kernel(
        _sc_body,
        out_type=jax.ShapeDtypeStruct((T, H), jnp.float32),
        mesh=mesh,
        compiler_params=pltpu.CompilerParams(needs_layout_passes=False),
        scratch_types=[
            pltpu.VMEM((CH,), jnp.int32),
            pltpu.VMEM((CH, H), jnp.float32),
            pltpu.VMEM((CH, H), jnp.float32),
            pltpu.VMEM((2, H), jnp.float32),
            pltpu.VMEM((H,), jnp.float32),
            pltpu.VMEM((H,), jnp.float32),
            pltpu.VMEM((CH,), jnp.int32),
            pltpu.SemaphoreType.DMA,
        ],
    )
    return f(ids, tts, word_emb, pos_emb, tok_type_emb, ln_gamma, ln_beta)


def kernel(input_ids, token_type_ids, word_emb, pos_emb, tok_type_emb,
           ln_gamma, ln_beta):
    ids = input_ids.reshape(T).astype(jnp.int32)
    tts = token_type_ids.reshape(T).astype(jnp.int32)
    out = _run(ids, tts, word_emb, pos_emb, tok_type_emb, ln_gamma, ln_beta)
    return out.reshape(B, S, H)


# speedup vs baseline: 2.8320x; 2.8320x over previous
import functools

import jax
import jax.numpy as jnp
from jax import lax
from jax.experimental import pallas as pl
from jax.experimental.pallas import tpu as pltpu
from jax.experimental.pallas import tpu_sc as plsc

VOCAB = 30522
H = 768
NC, NS = 2, 16
NW = NC * NS
B, S = 4, 2048
T = B * S
PER_W = T // NW
CH = 32
NCHUNK = PER_W // CH
NBUF = 4
EPS = 1e-12

BS = 256
NSB = S // BS


def _sc_gather_body(ids_hbm, word_hbm, out_hbm, *scratch):
    idxs = scratch[0:NBUF]
    bufs = scratch[NBUF:2 * NBUF]
    sem_g = scratch[2 * NBUF:3 * NBUF]
    sem_o = scratch[3 * NBUF:4 * NBUF]

    wid = lax.axis_index("s") * NC + lax.axis_index("c")
    base0 = wid * PER_W

    cps_g = [None] * NCHUNK
    cps_o = [None] * NCHUNK
    for c in range(NCHUNK):
        b = c % NBUF
        if c >= NBUF:
            cps_o[c - NBUF].wait()
        base = base0 + c * CH
        pltpu.sync_copy(ids_hbm.at[pl.ds(base, CH)], idxs[b])
        cps_g[c] = pltpu.async_copy(word_hbm.at[idxs[b]], bufs[b], sem_g[b])
        if c >= 1:
            pb = (c - 1) % NBUF
            cps_g[c - 1].wait()
            pbase = base0 + (c - 1) * CH
            cps_o[c - 1] = pltpu.async_copy(
                bufs[pb], out_hbm.at[pl.ds(pbase, CH)], sem_o[pb])
    c = NCHUNK - 1
    cps_g[c].wait()
    cps_o[c] = pltpu.async_copy(
        bufs[c % NBUF], out_hbm.at[pl.ds(base0 + c * CH, CH)],
        sem_o[c % NBUF])
    for c in range(max(0, NCHUNK - NBUF), NCHUNK):
        cps_o[c].wait()


def _sc_gather(ids, word_emb):
    mesh = plsc.VectorSubcoreMesh(core_axis_name="c", subcore_axis_name="s",
                                  num_cores=NC, num_subcores=NS)
    scratch = ([pltpu.VMEM((CH,), jnp.int32) for _ in range(NBUF)]
               + [pltpu.VMEM((CH, H), jnp.float32) for _ in range(NBUF)]
               + [pltpu.SemaphoreType.DMA for _ in range(2 * NBUF)])
    f = pl.kernel(
        _sc_gather_body,
        out_type=jax.ShapeDtypeStruct((T, H), jnp.float32),
        mesh=mesh,
        scratch_types=scratch,
    )
    return f(ids, word_emb)


def _tc_ln_body(x_ref, pos_ref, ttf_ref, tok_ref, gam_ref, bet_ref, o_ref):
    x = x_ref[...]
    t = ttf_ref[...]
    tok0 = tok_ref[0:1, :]
    tokrow = tok0 + t * (tok_ref[1:2, :] - tok0)
    x = x + pos_ref[...] + tokrow
    m = jnp.mean(x, axis=1, keepdims=True)
    xc = x - m
    var = jnp.mean(xc * xc, axis=1, keepdims=True)
    inv = lax.rsqrt(var + EPS)
    o_ref[...] = xc * inv * gam_ref[...] + bet_ref[...]


def _tc_ln(gathered, pos_emb, ttf, tok_type_emb, ln_gamma, ln_beta):
    grid = (NSB, B)
    return pl.pallas_call(
        _tc_ln_body,
        grid=grid,
        in_specs=[
            pl.BlockSpec((BS, H), lambda s, b: (b * NSB + s, 0)),
            pl.BlockSpec((BS, H), lambda s, b: (s, 0)),
            pl.BlockSpec((BS, 1), lambda s, b: (b * NSB + s, 0)),
            pl.BlockSpec((2, H), lambda s, b: (0, 0)),
            pl.BlockSpec((H,), lambda s, b: (0,)),
            pl.BlockSpec((H,), lambda s, b: (0,)),
        ],
        out_specs=pl.BlockSpec((BS, H), lambda s, b: (b * NSB + s, 0)),
        out_shape=jax.ShapeDtypeStruct((T, H), jnp.float32),
    )(gathered, pos_emb, ttf, tok_type_emb, ln_gamma, ln_beta)


@jax.jit
def _run(ids, ttf, word_emb, pos_emb, tok_type_emb, ln_gamma, ln_beta):
    gathered = _sc_gather(ids, word_emb)
    return _tc_ln(gathered, pos_emb, ttf, tok_type_emb, ln_gamma, ln_beta)


def kernel(input_ids, token_type_ids, word_emb, pos_emb, tok_type_emb,
           ln_gamma, ln_beta):
    ids = input_ids.reshape(T).astype(jnp.int32)
    ttf = token_type_ids.reshape(T, 1).astype(jnp.float32)
    out = _run(ids, ttf, word_emb, pos_emb, tok_type_emb, ln_gamma, ln_beta)
    return out.reshape(B, S, H)


# speedup vs baseline: 3.2754x; 1.1565x over previous
import functools

import jax
import jax.numpy as jnp
from jax import lax
from jax.experimental import pallas as pl
from jax.experimental.pallas import tpu as pltpu
from jax.experimental.pallas import tpu_sc as plsc

VOCAB = 30522
H = 768
NC, NS = 2, 16
NW = NC * NS
B, S = 4, 2048
T = B * S
PER_W = T // NW
CH = 32
NCHUNK = PER_W // CH
NBUF = 4
EPS = 1e-12

BS = 512
NSB = S // BS


def _sc_gather_body(ids_hbm, word_hbm, out_hbm, *scratch):
    idxs = scratch[0:NBUF]
    bufs = scratch[NBUF:2 * NBUF]
    sem_g = scratch[2 * NBUF:3 * NBUF]
    sem_o = scratch[3 * NBUF:4 * NBUF]

    wid = lax.axis_index("s") * NC + lax.axis_index("c")
    base0 = wid * PER_W

    cps_g = [None] * NCHUNK
    cps_o = [None] * NCHUNK
    for c in range(NCHUNK):
        b = c % NBUF
        if c >= NBUF:
            cps_o[c - NBUF].wait()
        base = base0 + c * CH
        pltpu.sync_copy(ids_hbm.at[pl.ds(base, CH)], idxs[b])
        cps_g[c] = pltpu.async_copy(word_hbm.at[idxs[b]], bufs[b], sem_g[b])
        if c >= 1:
            pb = (c - 1) % NBUF
            cps_g[c - 1].wait()
            pbase = base0 + (c - 1) * CH
            cps_o[c - 1] = pltpu.async_copy(
                bufs[pb], out_hbm.at[pl.ds(pbase, CH)], sem_o[pb])
    c = NCHUNK - 1
    cps_g[c].wait()
    cps_o[c] = pltpu.async_copy(
        bufs[c % NBUF], out_hbm.at[pl.ds(base0 + c * CH, CH)],
        sem_o[c % NBUF])
    for c in range(max(0, NCHUNK - NBUF), NCHUNK):
        cps_o[c].wait()


def _sc_gather(ids, word_emb):
    mesh = plsc.VectorSubcoreMesh(core_axis_name="c", subcore_axis_name="s",
                                  num_cores=NC, num_subcores=NS)
    scratch = ([pltpu.VMEM((CH,), jnp.int32) for _ in range(NBUF)]
               + [pltpu.VMEM((CH, H), jnp.float32) for _ in range(NBUF)]
               + [pltpu.SemaphoreType.DMA for _ in range(2 * NBUF)])
    f = pl.kernel(
        _sc_gather_body,
        out_type=jax.ShapeDtypeStruct((T, H), jnp.float32),
        mesh=mesh,
        scratch_types=scratch,
    )
    return f(ids, word_emb)


def _tc_ln_body(x_ref, pos_ref, ttf_ref, tok_ref, gam_ref, bet_ref, o_ref):
    x = x_ref[...]
    t = ttf_ref[...]
    tok0 = tok_ref[0:1, :]
    tokrow = tok0 + t * (tok_ref[1:2, :] - tok0)
    x = x + pos_ref[...] + tokrow
    m = jnp.mean(x, axis=1, keepdims=True)
    xc = x - m
    var = jnp.mean(xc * xc, axis=1, keepdims=True)
    inv = lax.rsqrt(var + EPS)
    o_ref[...] = xc * inv * gam_ref[...] + bet_ref[...]


def _tc_ln(gathered, pos_emb, ttf, tok_type_emb, ln_gamma, ln_beta):
    grid = (NSB, B)
    return pl.pallas_call(
        _tc_ln_body,
        grid=grid,
        in_specs=[
            pl.BlockSpec((BS, H), lambda s, b: (b * NSB + s, 0)),
            pl.BlockSpec((BS, H), lambda s, b: (s, 0)),
            pl.BlockSpec((BS, 1), lambda s, b: (b * NSB + s, 0)),
            pl.BlockSpec((2, H), lambda s, b: (0, 0)),
            pl.BlockSpec((H,), lambda s, b: (0,)),
            pl.BlockSpec((H,), lambda s, b: (0,)),
        ],
        out_specs=pl.BlockSpec((BS, H), lambda s, b: (b * NSB + s, 0)),
        out_shape=jax.ShapeDtypeStruct((T, H), jnp.float32),
    )(gathered, pos_emb, ttf, tok_type_emb, ln_gamma, ln_beta)


@jax.jit
def _run(ids, ttf, word_emb, pos_emb, tok_type_emb, ln_gamma, ln_beta):
    gathered = _sc_gather(ids, word_emb)
    return _tc_ln(gathered, pos_emb, ttf, tok_type_emb, ln_gamma, ln_beta)


def kernel(input_ids, token_type_ids, word_emb, pos_emb, tok_type_emb,
           ln_gamma, ln_beta):
    ids = input_ids.reshape(T).astype(jnp.int32)
    ttf = token_type_ids.reshape(T, 1).astype(jnp.float32)
    out = _run(ids, ttf, word_emb, pos_emb, tok_type_emb, ln_gamma, ln_beta)
    return out.reshape(B, S, H)


# speedup vs baseline: 3.4353x; 1.0488x over previous
import functools

import jax
import jax.numpy as jnp
from jax import lax
from jax.experimental import pallas as pl
from jax.experimental.pallas import tpu as pltpu
from jax.experimental.pallas import tpu_sc as plsc

VOCAB = 30522
H = 768
NC, NS = 2, 16
NW = NC * NS
B, S = 4, 2048
T = B * S
PER_W = T // NW
CH = 32
NCHUNK = PER_W // CH
NBUF = 4
EPS = 1e-12

BS = 1024
NSB = S // BS


def _sc_gather_body(ids_hbm, word_hbm, out_hbm, *scratch):
    idxs = scratch[0:NBUF]
    bufs = scratch[NBUF:2 * NBUF]
    sem_g = scratch[2 * NBUF:3 * NBUF]
    sem_o = scratch[3 * NBUF:4 * NBUF]

    wid = lax.axis_index("s") * NC + lax.axis_index("c")
    base0 = wid * PER_W

    cps_g = [None] * NCHUNK
    cps_o = [None] * NCHUNK
    for c in range(NCHUNK):
        b = c % NBUF
        if c >= NBUF:
            cps_o[c - NBUF].wait()
        base = base0 + c * CH
        pltpu.sync_copy(ids_hbm.at[pl.ds(base, CH)], idxs[b])
        cps_g[c] = pltpu.async_copy(word_hbm.at[idxs[b]], bufs[b], sem_g[b])
        if c >= 1:
            pb = (c - 1) % NBUF
            cps_g[c - 1].wait()
            pbase = base0 + (c - 1) * CH
            cps_o[c - 1] = pltpu.async_copy(
                bufs[pb], out_hbm.at[pl.ds(pbase, CH)], sem_o[pb])
    c = NCHUNK - 1
    cps_g[c].wait()
    cps_o[c] = pltpu.async_copy(
        bufs[c % NBUF], out_hbm.at[pl.ds(base0 + c * CH, CH)],
        sem_o[c % NBUF])
    for c in range(max(0, NCHUNK - NBUF), NCHUNK):
        cps_o[c].wait()


def _sc_gather(ids, word_emb):
    mesh = plsc.VectorSubcoreMesh(core_axis_name="c", subcore_axis_name="s",
                                  num_cores=NC, num_subcores=NS)
    scratch = ([pltpu.VMEM((CH,), jnp.int32) for _ in range(NBUF)]
               + [pltpu.VMEM((CH, H), jnp.float32) for _ in range(NBUF)]
               + [pltpu.SemaphoreType.DMA for _ in range(2 * NBUF)])
    f = pl.kernel(
        _sc_gather_body,
        out_type=jax.ShapeDtypeStruct((T, H), jnp.float32),
        mesh=mesh,
        scratch_types=scratch,
    )
    return f(ids, word_emb)


def _tc_ln_body(x_ref, pos_ref, ttf_ref, tok_ref, gam_ref, bet_ref, o_ref):
    x = x_ref[...]
    t = ttf_ref[...]
    tok0 = tok_ref[0:1, :]
    tokrow = tok0 + t * (tok_ref[1:2, :] - tok0)
    x = x + pos_ref[...] + tokrow
    m = jnp.mean(x, axis=1, keepdims=True)
    xc = x - m
    var = jnp.mean(xc * xc, axis=1, keepdims=True)
    inv = lax.rsqrt(var + EPS)
    o_ref[...] = xc * inv * gam_ref[...] + bet_ref[...]


def _tc_ln(gathered, pos_emb, ttf, tok_type_emb, ln_gamma, ln_beta):
    grid = (NSB, B)
    return pl.pallas_call(
        _tc_ln_body,
        grid=grid,
        in_specs=[
            pl.BlockSpec((BS, H), lambda s, b: (b * NSB + s, 0)),
            pl.BlockSpec((BS, H), lambda s, b: (s, 0)),
            pl.BlockSpec((BS, 1), lambda s, b: (b * NSB + s, 0)),
            pl.BlockSpec((2, H), lambda s, b: (0, 0)),
            pl.BlockSpec((H,), lambda s, b: (0,)),
            pl.BlockSpec((H,), lambda s, b: (0,)),
        ],
        out_specs=pl.BlockSpec((BS, H), lambda s, b: (b * NSB + s, 0)),
        out_shape=jax.ShapeDtypeStruct((T, H), jnp.float32),
    )(gathered, pos_emb, ttf, tok_type_emb, ln_gamma, ln_beta)


@jax.jit
def _run(ids, ttf, word_emb, pos_emb, tok_type_emb, ln_gamma, ln_beta):
    gathered = _sc_gather(ids, word_emb)
    return _tc_ln(gathered, pos_emb, ttf, tok_type_emb, ln_gamma, ln_beta)


def kernel(input_ids, token_type_ids, word_emb, pos_emb, tok_type_emb,
           ln_gamma, ln_beta):
    ids = input_ids.reshape(T).astype(jnp.int32)
    ttf = token_type_ids.reshape(T, 1).astype(jnp.float32)
    out = _run(ids, ttf, word_emb, pos_emb, tok_type_emb, ln_gamma, ln_beta)
    return out.reshape(B, S, H)
